# baseline (speedup 1.0000x reference)
import jax
import jax.numpy as jnp
from jax import lax
from jax.experimental import pallas as pl
from jax.experimental.pallas import tpu as pltpu
from jax.experimental.pallas import tpu_sc as plsc

N = 10000
E = 320000
F_IN = 128
HEADS = 8
HID = 8
D1 = HEADS * HID
NCLS = 16

NPAD = 10240
E_TOT = E + N
E_PAD = 331776
BLOCKS = 81
IDXROWS = 82
ROWS_PER_TILE = NPAD // 16

_f32 = jnp.float32
_i32 = jnp.int32


def _lane():
    return lax.iota(_i32, 16)


def _take(v, idx):
    dnums = lax.GatherDimensionNumbers(
        offset_dims=(), collapsed_slice_dims=(0,), start_index_map=(0,))
    return lax.gather(v, idx[:, None], dnums, (1,),
                      mode=lax.GatherScatterMode.PROMISE_IN_BOUNDS)


def _sc_layer1(src_hbm, dst_hbm, h1_hbm, a1_hbm, acc_hbm,
               idx_s, idx_d, a1s_b, a1d_b, h_b, msg_0, msg_1, acc_s,
               sem0, sem1, sem2, ssem0, ssem1):
    cid = lax.axis_index("c")
    sid = lax.axis_index("s")
    wid = cid * 16 + sid
    lane = _lane()
    swap8 = (lane + 8) % 16

    pltpu.sync_copy(src_hbm.at[wid], idx_s)
    pltpu.sync_copy(dst_hbm.at[wid], idx_d)

    @plsc.parallel_loop(0, 128, unroll=4)
    def _z(e):
        for k in range(5):
            msg_0[e, pl.ds(16 * k, 16)] = jnp.zeros((16,), _f32)
            msg_1[e, pl.ds(16 * k, 16)] = jnp.zeros((16,), _f32)
    base = sid * ROWS_PER_TILE
    for t in range(5):
        pltpu.sync_copy(msg_0, acc_s.at[pl.ds(base + 128 * t, 128)])
    plsc.subcore_barrier()
    pltpu.async_copy(msg_0, acc_s.at[idx_d.at[0]], ssem0, add=True)
    pltpu.async_copy(msg_1, acc_s.at[idx_d.at[0]], ssem1, add=True)

    def _outer(g, c):
        for q in range(2):
            mq = msg_0 if q == 0 else msg_1
            sq = ssem0 if q == 0 else ssem1
            j = 2 * g + q
            ir = idx_s.at[j]
            id_ = idx_d.at[j]
            c1 = pltpu.async_copy(h1_hbm.at[ir], h_b, sem0)
            c2 = pltpu.async_copy(a1_hbm.at[ir], a1s_b, sem1)
            c3 = pltpu.async_copy(a1_hbm.at[id_], a1d_b, sem2)
            c1.wait()
            c2.wait()
            c3.wait()
            pltpu.make_async_copy(mq, acc_s.at[id_], sq).wait()

            @plsc.parallel_loop(0, 128, unroll=8)
            def _edge(e):
                vs = a1s_b[e]
                vd = a1d_b[e]
                al = vs + _take(vd, swap8)
                al = jnp.where(al > 0, al, 0.2 * al)
                ex = jnp.exp(al)
                mq[e, pl.ds(64, 16)] = jnp.where(lane < 8, ex, 0.0)
                for k in range(4):
                    coef = _take(ex, 2 * k + jnp.where(lane < 8, 0, 1))
                    mq[e, pl.ds(16 * k, 16)] = h_b[e, pl.ds(16 * k, 16)] * coef

            pltpu.async_copy(mq, acc_s.at[id_], sq, add=True)
        return c
    lax.fori_loop(0, (BLOCKS + 1) // 2, _outer, 0)
    pltpu.make_async_copy(msg_0, acc_s.at[idx_d.at[0]], ssem0).wait()
    pltpu.make_async_copy(msg_1, acc_s.at[idx_d.at[0]], ssem1).wait()
    plsc.subcore_barrier()

    for t in range(5):
        r = base + 128 * t
        pltpu.sync_copy(acc_s.at[pl.ds(r, 128)], acc_hbm.at[cid, pl.ds(r, 128)])


def _sc_layer2(src_hbm, dst_hbm, h2_hbm, a2_hbm, acc_hbm,
               idx_s, idx_d, a2s_b, a2d_b, h_b, msg_0, msg_1, acc_s,
               sem0, sem1, sem2, ssem0, ssem1):
    cid = lax.axis_index("c")
    sid = lax.axis_index("s")
    wid = cid * 16 + sid
    lane = _lane()
    rot1 = (lane + 1) % 16
    zero16 = lane * 0

    pltpu.sync_copy(src_hbm.at[wid], idx_s)
    pltpu.sync_copy(dst_hbm.at[wid], idx_d)

    @plsc.parallel_loop(0, 128, unroll=4)
    def _z(e):
        for k in range(2):
            msg_0[e, pl.ds(16 * k, 16)] = jnp.zeros((16,), _f32)
            msg_1[e, pl.ds(16 * k, 16)] = jnp.zeros((16,), _f32)
    base = sid * ROWS_PER_TILE
    for t in range(5):
        pltpu.sync_copy(msg_0, acc_s.at[pl.ds(base + 128 * t, 128)])
    plsc.subcore_barrier()
    pltpu.async_copy(msg_0, acc_s.at[idx_d.at[0]], ssem0, add=True)
    pltpu.async_copy(msg_1, acc_s.at[idx_d.at[0]], ssem1, add=True)

    def _outer(g, c):
        for q in range(2):
            mq = msg_0 if q == 0 else msg_1
            sq = ssem0 if q == 0 else ssem1
            j = 2 * g + q
            ir = idx_s.at[j]
            id_ = idx_d.at[j]
            c1 = pltpu.async_copy(h2_hbm.at[ir], h_b, sem0)
            c2 = pltpu.async_copy(a2_hbm.at[ir], a2s_b, sem1)
            c3 = pltpu.async_copy(a2_hbm.at[id_], a2d_b, sem2)
            c1.wait()
            c2.wait()
            c3.wait()
            pltpu.make_async_copy(mq, acc_s.at[id_], sq).wait()

            @plsc.parallel_loop(0, 128, unroll=8)
            def _edge(e):
                vs = a2s_b[e]
                vd = a2d_b[e]
                t0 = vs + _take(vd, rot1)
                t0 = jnp.where(t0 > 0, t0, 0.2 * t0)
                ex = jnp.exp(t0)
                coef = _take(ex, zero16)
                mq[e, pl.ds(0, 16)] = h_b[e] * coef
                mq[e, pl.ds(16, 16)] = jnp.where(lane == 0, coef, 0.0)

            pltpu.async_copy(mq, acc_s.at[id_], sq, add=True)
        return c
    lax.fori_loop(0, (BLOCKS + 1) // 2, _outer, 0)
    pltpu.make_async_copy(msg_0, acc_s.at[idx_d.at[0]], ssem0).wait()
    pltpu.make_async_copy(msg_1, acc_s.at[idx_d.at[0]], ssem1).wait()
    plsc.subcore_barrier()

    for t in range(5):
        r = base + 128 * t
        pltpu.sync_copy(acc_s.at[pl.ds(r, 128)], acc_hbm.at[cid, pl.ds(r, 128)])


def _tc_a(x_ref, w1_ref, aall_ref, h_ref, a1_ref):
    h = jnp.dot(x_ref[...], w1_ref[...], preferred_element_type=_f32)
    h_ref[...] = h
    a1_ref[...] = jnp.dot(h, aall_ref[...], preferred_element_type=_f32)


def _tc_c(acc_ref, b1_ref, w2_ref, acomb_ref, h2_ref, a2_ref):
    acc = acc_ref[0] + acc_ref[1]
    den = acc[:, D1:D1 + HEADS] + 1e-16
    blk = acc.shape[0]
    msg = acc[:, :D1].reshape(blk, HEADS, HID)
    out1 = msg / den[:, :, None] + b1_ref[...].reshape(1, HEADS, HID)
    out1 = out1.reshape(blk, D1)
    out1 = jnp.where(out1 > 0, out1, jnp.exp(jnp.minimum(out1, 0.0)) - 1.0)
    h2 = jnp.dot(out1, w2_ref[...], preferred_element_type=_f32)
    h2_ref[...] = h2
    a2_ref[...] = jnp.dot(h2, acomb_ref[...], preferred_element_type=_f32)


def _tc_e(acc_ref, b2_ref, out_ref):
    num = acc_ref[0, :, :NCLS] + acc_ref[1, :, :NCLS]
    den = acc_ref[0, :, NCLS:NCLS + 1] + acc_ref[1, :, NCLS:NCLS + 1] + 1e-16
    out_ref[...] = num / den + b2_ref[...]


def kernel(x, edge_index, edge_attr, W1, att_src1, att_dst1, b1,
           W2, att_src2, att_dst2, b2):
    x = x.astype(_f32)
    loop = jnp.arange(N, dtype=_i32)
    src = jnp.concatenate([edge_index[0].astype(_i32), loop])
    dst = jnp.concatenate([edge_index[1].astype(_i32), loop])
    padlen = E_PAD - E_TOT
    pad_idx = jnp.full((padlen,), N, _i32)
    tailpad = jnp.full((32, IDXROWS - BLOCKS, 128), N, _i32)
    src2d = jnp.concatenate(
        [jnp.concatenate([src, pad_idx]).reshape(32, BLOCKS, 128), tailpad], axis=1)
    dst2d = jnp.concatenate(
        [jnp.concatenate([dst, pad_idx]).reshape(32, BLOCKS, 128), tailpad], axis=1)

    xp = jnp.zeros((NPAD, F_IN), _f32).at[:N].set(x)

    r = jnp.arange(D1)
    aall1 = (jnp.zeros((D1, 16), _f32)
             .at[r, r // HID].set(att_src1.reshape(D1).astype(_f32))
             .at[r, r // HID + HEADS].set(att_dst1.reshape(D1).astype(_f32)))
    acomb2 = (jnp.zeros((NCLS, 16), _f32)
              .at[:, 0].set(att_src2.reshape(NCLS).astype(_f32))
              .at[:, 1].set(att_dst2.reshape(NCLS).astype(_f32)))

    BLK = 1024
    grid = NPAD // BLK

    h1, a1 = pl.pallas_call(
        _tc_a,
        grid=(grid,),
        in_specs=[
            pl.BlockSpec((BLK, F_IN), lambda i: (i, 0)),
            pl.BlockSpec((F_IN, D1), lambda i: (0, 0)),
            pl.BlockSpec((D1, 16), lambda i: (0, 0)),
        ],
        out_specs=[
            pl.BlockSpec((BLK, D1), lambda i: (i, 0)),
            pl.BlockSpec((BLK, 16), lambda i: (i, 0)),
        ],
        out_shape=[
            jax.ShapeDtypeStruct((NPAD, D1), _f32),
            jax.ShapeDtypeStruct((NPAD, 16), _f32),
        ],
    )(xp, W1.astype(_f32), aall1)

    mesh = plsc.VectorSubcoreMesh(core_axis_name="c", subcore_axis_name="s")

    acc1 = pl.kernel(
        _sc_layer1,
        out_type=jax.ShapeDtypeStruct((2, NPAD, 80), _f32),
        mesh=mesh,
        compiler_params=pltpu.CompilerParams(use_tc_tiling_on_sc=False),
        scratch_types=[
            pltpu.VMEM((IDXROWS, 128), _i32),
            pltpu.VMEM((IDXROWS, 128), _i32),
            pltpu.VMEM((128, 16), _f32),
            pltpu.VMEM((128, 16), _f32),
            pltpu.VMEM((128, D1), _f32),
            pltpu.VMEM((128, 80), _f32),
            pltpu.VMEM((128, 80), _f32),
            pltpu.VMEM_SHARED((NPAD, 80), _f32),
            pltpu.SemaphoreType.DMA,
            pltpu.SemaphoreType.DMA,
            pltpu.SemaphoreType.DMA,
            pltpu.SemaphoreType.DMA,
            pltpu.SemaphoreType.DMA,
        ],
    )(src2d, dst2d, h1, a1)

    h2, a2 = pl.pallas_call(
        _tc_c,
        grid=(grid,),
        in_specs=[
            pl.BlockSpec((2, BLK, 80), lambda i: (0, i, 0)),
            pl.BlockSpec((1, D1), lambda i: (0, 0)),
            pl.BlockSpec((D1, NCLS), lambda i: (0, 0)),
            pl.BlockSpec((NCLS, 16), lambda i: (0, 0)),
        ],
        out_specs=[
            pl.BlockSpec((BLK, NCLS), lambda i: (i, 0)),
            pl.BlockSpec((BLK, 16), lambda i: (i, 0)),
        ],
        out_shape=[
            jax.ShapeDtypeStruct((NPAD, NCLS), _f32),
            jax.ShapeDtypeStruct((NPAD, 16), _f32),
        ],
    )(acc1, b1.reshape(1, D1).astype(_f32), W2.astype(_f32), acomb2)

    acc2 = pl.kernel(
        _sc_layer2,
        out_type=jax.ShapeDtypeStruct((2, NPAD, 32), _f32),
        mesh=mesh,
        compiler_params=pltpu.CompilerParams(use_tc_tiling_on_sc=False),
        scratch_types=[
            pltpu.VMEM((IDXROWS, 128), _i32),
            pltpu.VMEM((IDXROWS, 128), _i32),
            pltpu.VMEM((128, 16), _f32),
            pltpu.VMEM((128, 16), _f32),
            pltpu.VMEM((128, 16), _f32),
            pltpu.VMEM((128, 32), _f32),
            pltpu.VMEM((128, 32), _f32),
            pltpu.VMEM_SHARED((NPAD, 32), _f32),
            pltpu.SemaphoreType.DMA,
            pltpu.SemaphoreType.DMA,
            pltpu.SemaphoreType.DMA,
            pltpu.SemaphoreType.DMA,
            pltpu.SemaphoreType.DMA,
        ],
    )(src2d, dst2d, h2, a2)

    out = pl.pallas_call(
        _tc_e,
        grid=(grid,),
        in_specs=[
            pl.BlockSpec((2, BLK, 32), lambda i: (0, i, 0)),
            pl.BlockSpec((1, NCLS), lambda i: (0, 0)),
        ],
        out_specs=pl.BlockSpec((BLK, NCLS), lambda i: (i, 0)),
        out_shape=jax.ShapeDtypeStruct((NPAD, NCLS), _f32),
    )(acc2, b2.reshape(1, NCLS).astype(_f32))

    return out[:N]


# speedup vs baseline: 1.0906x; 1.0906x over previous
import jax
import jax.numpy as jnp
from jax import lax
from jax.experimental import pallas as pl
from jax.experimental.pallas import tpu as pltpu
from jax.experimental.pallas import tpu_sc as plsc

N = 10000
E = 320000
F_IN = 128
HEADS = 8
HID = 8
D1 = HEADS * HID
NCLS = 16

NPAD = 10240
E_TOT = E + N
E_PAD = 331776
BLOCKS = 81
ROWS_PER_TILE = NPAD // 16

_f32 = jnp.float32
_i32 = jnp.int32


def _lane():
    return lax.iota(_i32, 16)


def _take(v, idx):
    dnums = lax.GatherDimensionNumbers(
        offset_dims=(), collapsed_slice_dims=(0,), start_index_map=(0,))
    return lax.gather(v, idx[:, None], dnums, (1,),
                      mode=lax.GatherScatterMode.PROMISE_IN_BOUNDS)


def _sc_layer1(src_hbm, dst_hbm, h1x_hbm, a1_hbm, acc_hbm,
               idx_s, idx_d, a1d_b, h_b, msg_b, acc_s,
               sem0, sem2):
    cid = lax.axis_index("c")
    sid = lax.axis_index("s")
    wid = cid * 16 + sid
    lane = _lane()
    swap8 = (lane + 8) % 16

    pltpu.sync_copy(src_hbm.at[wid], idx_s)
    pltpu.sync_copy(dst_hbm.at[wid], idx_d)

    @plsc.parallel_loop(0, 128, unroll=4)
    def _z(e):
        for k in range(5):
            msg_b[e, pl.ds(16 * k, 16)] = jnp.zeros((16,), _f32)
    base = sid * ROWS_PER_TILE
    for t in range(5):
        pltpu.sync_copy(msg_b, acc_s.at[pl.ds(base + 128 * t, 128)])
    plsc.subcore_barrier()

    def _block(j, c):
        ir = idx_s.at[j]
        id_ = idx_d.at[j]
        c1 = pltpu.async_copy(h1x_hbm.at[ir], h_b, sem0)
        c3 = pltpu.async_copy(a1_hbm.at[id_], a1d_b, sem2)
        c1.wait()
        c3.wait()

        @plsc.parallel_loop(0, 128, unroll=8)
        def _edge(e):
            vs = h_b[e, pl.ds(64, 16)]
            vd = a1d_b[e]
            al = vs + _take(vd, swap8)
            al = jnp.where(al > 0, al, 0.2 * al)
            ex = jnp.exp(al)
            msg_b[e, pl.ds(64, 16)] = jnp.where(lane < 8, ex, 0.0)
            for k in range(4):
                coef = _take(ex, 2 * k + jnp.where(lane < 8, 0, 1))
                msg_b[e, pl.ds(16 * k, 16)] = h_b[e, pl.ds(16 * k, 16)] * coef

        pltpu.sync_copy(msg_b, acc_s.at[id_], add=True)
        return c
    lax.fori_loop(0, BLOCKS, _block, 0)
    plsc.subcore_barrier()

    for t in range(5):
        r = base + 128 * t
        pltpu.sync_copy(acc_s.at[pl.ds(r, 128)], acc_hbm.at[cid, pl.ds(r, 128)])


def _sc_layer2(src_hbm, dst_hbm, h2x_hbm, a2_hbm, acc_hbm,
               idx_s, idx_d, a2d_b, h_b, msg_b, acc_s,
               sem0, sem2):
    cid = lax.axis_index("c")
    sid = lax.axis_index("s")
    wid = cid * 16 + sid
    lane = _lane()
    rot1 = (lane + 1) % 16
    zero16 = lane * 0

    pltpu.sync_copy(src_hbm.at[wid], idx_s)
    pltpu.sync_copy(dst_hbm.at[wid], idx_d)

    @plsc.parallel_loop(0, 128, unroll=4)
    def _z(e):
        for k in range(2):
            msg_b[e, pl.ds(16 * k, 16)] = jnp.zeros((16,), _f32)
    base = sid * ROWS_PER_TILE
    for t in range(5):
        pltpu.sync_copy(msg_b, acc_s.at[pl.ds(base + 128 * t, 128)])
    plsc.subcore_barrier()

    def _block(j, c):
        ir = idx_s.at[j]
        id_ = idx_d.at[j]
        c1 = pltpu.async_copy(h2x_hbm.at[ir], h_b, sem0)
        c3 = pltpu.async_copy(a2_hbm.at[id_], a2d_b, sem2)
        c1.wait()
        c3.wait()

        @plsc.parallel_loop(0, 128, unroll=8)
        def _edge(e):
            vs = h_b[e, pl.ds(16, 16)]
            vd = a2d_b[e]
            t0 = vs + _take(vd, rot1)
            t0 = jnp.where(t0 > 0, t0, 0.2 * t0)
            ex = jnp.exp(t0)
            coef = _take(ex, zero16)
            msg_b[e, pl.ds(0, 16)] = h_b[e, pl.ds(0, 16)] * coef
            msg_b[e, pl.ds(16, 16)] = jnp.where(lane == 0, coef, 0.0)

        pltpu.sync_copy(msg_b, acc_s.at[id_], add=True)
        return c
    lax.fori_loop(0, BLOCKS, _block, 0)
    plsc.subcore_barrier()

    for t in range(5):
        r = base + 128 * t
        pltpu.sync_copy(acc_s.at[pl.ds(r, 128)], acc_hbm.at[cid, pl.ds(r, 128)])


def _tc_a(x_ref, w1_ref, aall_ref, h_ref, a1_ref):
    h = jnp.dot(x_ref[...], w1_ref[...], preferred_element_type=_f32)
    a1 = jnp.dot(h, aall_ref[...], preferred_element_type=_f32)
    h_ref[...] = jnp.concatenate([h, a1], axis=1)
    a1_ref[...] = a1


def _tc_c(acc_ref, b1_ref, w2_ref, acomb_ref, h2_ref, a2_ref):
    acc = acc_ref[0] + acc_ref[1]
    den = acc[:, D1:D1 + HEADS] + 1e-16
    blk = acc.shape[0]
    msg = acc[:, :D1].reshape(blk, HEADS, HID)
    out1 = msg / den[:, :, None] + b1_ref[...].reshape(1, HEADS, HID)
    out1 = out1.reshape(blk, D1)
    out1 = jnp.where(out1 > 0, out1, jnp.exp(jnp.minimum(out1, 0.0)) - 1.0)
    h2 = jnp.dot(out1, w2_ref[...], preferred_element_type=_f32)
    a2 = jnp.dot(h2, acomb_ref[...], preferred_element_type=_f32)
    h2_ref[...] = jnp.concatenate([h2, a2], axis=1)
    a2_ref[...] = a2


def _tc_e(acc_ref, b2_ref, out_ref):
    num = acc_ref[0, :, :NCLS] + acc_ref[1, :, :NCLS]
    den = acc_ref[0, :, NCLS:NCLS + 1] + acc_ref[1, :, NCLS:NCLS + 1] + 1e-16
    out_ref[...] = num / den + b2_ref[...]


def kernel(x, edge_index, edge_attr, W1, att_src1, att_dst1, b1,
           W2, att_src2, att_dst2, b2):
    x = x.astype(_f32)
    loop = jnp.arange(N, dtype=_i32)
    src = jnp.concatenate([edge_index[0].astype(_i32), loop])
    dst = jnp.concatenate([edge_index[1].astype(_i32), loop])
    padlen = E_PAD - E_TOT
    pad_idx = jnp.full((padlen,), N, _i32)
    src2d = jnp.concatenate([src, pad_idx]).reshape(32, BLOCKS, 128)
    dst2d = jnp.concatenate([dst, pad_idx]).reshape(32, BLOCKS, 128)

    xp = jnp.zeros((NPAD, F_IN), _f32).at[:N].set(x)

    r = jnp.arange(D1)
    aall1 = (jnp.zeros((D1, 16), _f32)
             .at[r, r // HID].set(att_src1.reshape(D1).astype(_f32))
             .at[r, r // HID + HEADS].set(att_dst1.reshape(D1).astype(_f32)))
    acomb2 = (jnp.zeros((NCLS, 16), _f32)
              .at[:, 0].set(att_src2.reshape(NCLS).astype(_f32))
              .at[:, 1].set(att_dst2.reshape(NCLS).astype(_f32)))

    BLK = 1024
    grid = NPAD // BLK

    h1, a1 = pl.pallas_call(
        _tc_a,
        grid=(grid,),
        in_specs=[
            pl.BlockSpec((BLK, F_IN), lambda i: (i, 0)),
            pl.BlockSpec((F_IN, D1), lambda i: (0, 0)),
            pl.BlockSpec((D1, 16), lambda i: (0, 0)),
        ],
        out_specs=[
            pl.BlockSpec((BLK, D1 + 16), lambda i: (i, 0)),
            pl.BlockSpec((BLK, 16), lambda i: (i, 0)),
        ],
        out_shape=[
            jax.ShapeDtypeStruct((NPAD, D1 + 16), _f32),
            jax.ShapeDtypeStruct((NPAD, 16), _f32),
        ],
    )(xp, W1.astype(_f32), aall1)

    mesh = plsc.VectorSubcoreMesh(core_axis_name="c", subcore_axis_name="s")

    acc1 = pl.kernel(
        _sc_layer1,
        out_type=jax.ShapeDtypeStruct((2, NPAD, 80), _f32),
        mesh=mesh,
        compiler_params=pltpu.CompilerParams(use_tc_tiling_on_sc=False),
        scratch_types=[
            pltpu.VMEM((BLOCKS, 128), _i32),
            pltpu.VMEM((BLOCKS, 128), _i32),
            pltpu.VMEM((128, 16), _f32),
            pltpu.VMEM((128, D1 + 16), _f32),
            pltpu.VMEM((128, 80), _f32),
            pltpu.VMEM_SHARED((NPAD, 80), _f32),
            pltpu.SemaphoreType.DMA,
            pltpu.SemaphoreType.DMA,
        ],
    )(src2d, dst2d, h1, a1)

    h2, a2 = pl.pallas_call(
        _tc_c,
        grid=(grid,),
        in_specs=[
            pl.BlockSpec((2, BLK, 80), lambda i: (0, i, 0)),
            pl.BlockSpec((1, D1), lambda i: (0, 0)),
            pl.BlockSpec((D1, NCLS), lambda i: (0, 0)),
            pl.BlockSpec((NCLS, 16), lambda i: (0, 0)),
        ],
        out_specs=[
            pl.BlockSpec((BLK, NCLS + 16), lambda i: (i, 0)),
            pl.BlockSpec((BLK, 16), lambda i: (i, 0)),
        ],
        out_shape=[
            jax.ShapeDtypeStruct((NPAD, NCLS + 16), _f32),
            jax.ShapeDtypeStruct((NPAD, 16), _f32),
        ],
    )(acc1, b1.reshape(1, D1).astype(_f32), W2.astype(_f32), acomb2)

    acc2 = pl.kernel(
        _sc_layer2,
        out_type=jax.ShapeDtypeStruct((2, NPAD, 32), _f32),
        mesh=mesh,
        compiler_params=pltpu.CompilerParams(use_tc_tiling_on_sc=False),
        scratch_types=[
            pltpu.VMEM((BLOCKS, 128), _i32),
            pltpu.VMEM((BLOCKS, 128), _i32),
            pltpu.VMEM((128, 16), _f32),
            pltpu.VMEM((128, NCLS + 16), _f32),
            pltpu.VMEM((128, 32), _f32),
            pltpu.VMEM_SHARED((NPAD, 32), _f32),
            pltpu.SemaphoreType.DMA,
            pltpu.SemaphoreType.DMA,
        ],
    )(src2d, dst2d, h2, a2)

    out = pl.pallas_call(
        _tc_e,
        grid=(grid,),
        in_specs=[
            pl.BlockSpec((2, BLK, 32), lambda i: (0, i, 0)),
            pl.BlockSpec((1, NCLS), lambda i: (0, 0)),
        ],
        out_specs=pl.BlockSpec((BLK, NCLS), lambda i: (i, 0)),
        out_shape=jax.ShapeDtypeStruct((NPAD, NCLS), _f32),
    )(acc2, b2.reshape(1, NCLS).astype(_f32))

    return out[:N]


# speedup vs baseline: 1.1382x; 1.0436x over previous
import jax
import jax.numpy as jnp
from jax import lax
from jax.experimental import pallas as pl
from jax.experimental.pallas import tpu as pltpu
from jax.experimental.pallas import tpu_sc as plsc

N = 10000
E = 320000
F_IN = 128
HEADS = 8
HID = 8
D1 = HEADS * HID
NCLS = 16

NPAD = 10240
E_TOT = E + N
E_PAD = 331776
BLOCKS = 81
ROWS_PER_TILE = NPAD // 16

_f32 = jnp.float32
_i32 = jnp.int32


def _lane():
    return lax.iota(_i32, 16)


def _take(v, idx):
    dnums = lax.GatherDimensionNumbers(
        offset_dims=(), collapsed_slice_dims=(0,), start_index_map=(0,))
    return lax.gather(v, idx[:, None], dnums, (1,),
                      mode=lax.GatherScatterMode.PROMISE_IN_BOUNDS)


def _sc_layer1(src_hbm, dst_hbm, h1_hbm, a1_hbm, acc_hbm,
               idx_s, idx_d, a1s_b, a1d_b, h_b, msg_b, acc_s,
               sem0, sem1, sem2):
    cid = lax.axis_index("c")
    sid = lax.axis_index("s")
    wid = cid * 16 + sid
    lane = _lane()
    swap8 = (lane + 8) % 16

    pltpu.sync_copy(src_hbm.at[wid], idx_s)
    pltpu.sync_copy(dst_hbm.at[wid], idx_d)

    @plsc.parallel_loop(0, 128, unroll=4)
    def _z(e):
        for k in range(5):
            msg_b[e, pl.ds(16 * k, 16)] = jnp.zeros((16,), _f32)
    base = sid * ROWS_PER_TILE
    for t in range(5):
        pltpu.sync_copy(msg_b, acc_s.at[pl.ds(base + 128 * t, 128)])
    plsc.subcore_barrier()

    def _block(j, c):
        ir = idx_s.at[j]
        id_ = idx_d.at[j]
        c1 = pltpu.async_copy(h1_hbm.at[ir], h_b, sem0)
        c2 = pltpu.async_copy(a1_hbm.at[ir], a1s_b, sem1)
        c3 = pltpu.async_copy(a1_hbm.at[id_], a1d_b, sem2)
        c1.wait()
        c2.wait()
        c3.wait()

        @plsc.parallel_loop(0, 128, unroll=8)
        def _edge(e):
            vs = a1s_b[e]
            vd = a1d_b[e]
            al = vs + _take(vd, swap8)
            al = jnp.where(al > 0, al, 0.2 * al)
            ex = jnp.exp(al)
            msg_b[e, pl.ds(64, 16)] = jnp.where(lane < 8, ex, 0.0)
            for k in range(4):
                coef = _take(ex, 2 * k + jnp.where(lane < 8, 0, 1))
                msg_b[e, pl.ds(16 * k, 16)] = h_b[e, pl.ds(16 * k, 16)] * coef

        pltpu.sync_copy(msg_b, acc_s.at[id_], add=True)
        return c
    lax.fori_loop(0, BLOCKS, _block, 0)
    plsc.subcore_barrier()

    for t in range(5):
        r = base + 128 * t
        pltpu.sync_copy(acc_s.at[pl.ds(r, 128)], acc_hbm.at[cid, pl.ds(r, 128)])


def _sc_layer2(src_hbm, dst_hbm, h2_hbm, a2_hbm, acc_hbm,
               idx_s, idx_d, a2s_b, a2d_b, h_b, msg_b, acc_s,
               sem0, sem1, sem2):
    cid = lax.axis_index("c")
    sid = lax.axis_index("s")
    wid = cid * 16 + sid
    lane = _lane()
    rot1 = (lane + 1) % 16
    zero16 = lane * 0

    pltpu.sync_copy(src_hbm.at[wid], idx_s)
    pltpu.sync_copy(dst_hbm.at[wid], idx_d)

    @plsc.parallel_loop(0, 128, unroll=4)
    def _z(e):
        for k in range(2):
            msg_b[e, pl.ds(16 * k, 16)] = jnp.zeros((16,), _f32)
    base = sid * ROWS_PER_TILE
    for t in range(5):
        pltpu.sync_copy(msg_b, acc_s.at[pl.ds(base + 128 * t, 128)])
    plsc.subcore_barrier()

    def _block(j, c):
        ir = idx_s.at[j]
        id_ = idx_d.at[j]
        c1 = pltpu.async_copy(h2_hbm.at[ir], h_b, sem0)
        c2 = pltpu.async_copy(a2_hbm.at[ir], a2s_b, sem1)
        c3 = pltpu.async_copy(a2_hbm.at[id_], a2d_b, sem2)
        c1.wait()
        c2.wait()
        c3.wait()

        @plsc.parallel_loop(0, 128, unroll=8)
        def _edge(e):
            vs = a2s_b[e]
            vd = a2d_b[e]
            t0 = vs + _take(vd, rot1)
            t0 = jnp.where(t0 > 0, t0, 0.2 * t0)
            ex = jnp.exp(t0)
            coef = _take(ex, zero16)
            msg_b[e, pl.ds(0, 16)] = h_b[e] * coef
            msg_b[e, pl.ds(16, 16)] = jnp.where(lane == 0, coef, 0.0)

        pltpu.sync_copy(msg_b, acc_s.at[id_], add=True)
        return c
    lax.fori_loop(0, BLOCKS, _block, 0)
    plsc.subcore_barrier()

    for t in range(5):
        r = base + 128 * t
        pltpu.sync_copy(acc_s.at[pl.ds(r, 128)], acc_hbm.at[cid, pl.ds(r, 128)])


def _tc_a(x_ref, w1_ref, aall_ref, h_ref, a1_ref):
    h = jnp.dot(x_ref[...], w1_ref[...], preferred_element_type=_f32)
    h_ref[...] = h
    a1_ref[...] = jnp.dot(h, aall_ref[...], preferred_element_type=_f32)


def _tc_c(acc_ref, b1_ref, w2_ref, acomb_ref, h2_ref, a2_ref):
    acc = acc_ref[0] + acc_ref[1]
    den = acc[:, D1:D1 + HEADS] + 1e-16
    blk = acc.shape[0]
    msg = acc[:, :D1].reshape(blk, HEADS, HID)
    out1 = msg / den[:, :, None] + b1_ref[...].reshape(1, HEADS, HID)
    out1 = out1.reshape(blk, D1)
    out1 = jnp.where(out1 > 0, out1, jnp.exp(jnp.minimum(out1, 0.0)) - 1.0)
    h2 = jnp.dot(out1, w2_ref[...], preferred_element_type=_f32)
    h2_ref[...] = h2
    a2_ref[...] = jnp.dot(h2, acomb_ref[...], preferred_element_type=_f32)


def _tc_e(acc_ref, b2_ref, out_ref):
    num = acc_ref[0, :, :NCLS] + acc_ref[1, :, :NCLS]
    den = acc_ref[0, :, NCLS:NCLS + 1] + acc_ref[1, :, NCLS:NCLS + 1] + 1e-16
    out_ref[...] = num / den + b2_ref[...]


def kernel(x, edge_index, edge_attr, W1, att_src1, att_dst1, b1,
           W2, att_src2, att_dst2, b2):
    x = x.astype(_f32)
    loop = jnp.arange(N, dtype=_i32)
    src = jnp.concatenate([edge_index[0].astype(_i32), loop])
    dst = jnp.concatenate([edge_index[1].astype(_i32), loop])
    padlen = E_PAD - E_TOT
    pad_idx = jnp.full((padlen,), N, _i32)
    src2d = jnp.concatenate([src, pad_idx]).reshape(32, BLOCKS, 128)
    dst2d = jnp.concatenate([dst, pad_idx]).reshape(32, BLOCKS, 128)

    xp = jnp.zeros((NPAD, F_IN), _f32).at[:N].set(x)

    r = jnp.arange(D1)
    aall1 = (jnp.zeros((D1, 16), _f32)
             .at[r, r // HID].set(att_src1.reshape(D1).astype(_f32))
             .at[r, r // HID + HEADS].set(att_dst1.reshape(D1).astype(_f32)))
    acomb2 = (jnp.zeros((NCLS, 16), _f32)
              .at[:, 0].set(att_src2.reshape(NCLS).astype(_f32))
              .at[:, 1].set(att_dst2.reshape(NCLS).astype(_f32)))

    BLK = 1024
    grid = NPAD // BLK

    h1, a1 = pl.pallas_call(
        _tc_a,
        grid=(grid,),
        in_specs=[
            pl.BlockSpec((BLK, F_IN), lambda i: (i, 0)),
            pl.BlockSpec((F_IN, D1), lambda i: (0, 0)),
            pl.BlockSpec((D1, 16), lambda i: (0, 0)),
        ],
        out_specs=[
            pl.BlockSpec((BLK, D1), lambda i: (i, 0)),
            pl.BlockSpec((BLK, 16), lambda i: (i, 0)),
        ],
        out_shape=[
            jax.ShapeDtypeStruct((NPAD, D1), _f32),
            jax.ShapeDtypeStruct((NPAD, 16), _f32),
        ],
    )(xp, W1.astype(_f32), aall1)

    mesh = plsc.VectorSubcoreMesh(core_axis_name="c", subcore_axis_name="s")

    acc1 = pl.kernel(
        _sc_layer1,
        out_type=jax.ShapeDtypeStruct((2, NPAD, 80), _f32),
        mesh=mesh,
        compiler_params=pltpu.CompilerParams(use_tc_tiling_on_sc=False),
        scratch_types=[
            pltpu.VMEM((BLOCKS, 128), _i32),
            pltpu.VMEM((BLOCKS, 128), _i32),
            pltpu.VMEM((128, 16), _f32),
            pltpu.VMEM((128, 16), _f32),
            pltpu.VMEM((128, D1), _f32),
            pltpu.VMEM((128, 80), _f32),
            pltpu.VMEM_SHARED((NPAD, 80), _f32),
            pltpu.SemaphoreType.DMA,
            pltpu.SemaphoreType.DMA,
            pltpu.SemaphoreType.DMA,
        ],
    )(src2d, dst2d, h1, a1)

    h2, a2 = pl.pallas_call(
        _tc_c,
        grid=(grid,),
        in_specs=[
            pl.BlockSpec((2, BLK, 80), lambda i: (0, i, 0)),
            pl.BlockSpec((1, D1), lambda i: (0, 0)),
            pl.BlockSpec((D1, NCLS), lambda i: (0, 0)),
            pl.BlockSpec((NCLS, 16), lambda i: (0, 0)),
        ],
        out_specs=[
            pl.BlockSpec((BLK, NCLS), lambda i: (i, 0)),
            pl.BlockSpec((BLK, 16), lambda i: (i, 0)),
        ],
        out_shape=[
            jax.ShapeDtypeStruct((NPAD, NCLS), _f32),
            jax.ShapeDtypeStruct((NPAD, 16), _f32),
        ],
    )(acc1, b1.reshape(1, D1).astype(_f32), W2.astype(_f32), acomb2)

    acc2 = pl.kernel(
        _sc_layer2,
        out_type=jax.ShapeDtypeStruct((2, NPAD, 32), _f32),
        mesh=mesh,
        compiler_params=pltpu.CompilerParams(use_tc_tiling_on_sc=False),
        scratch_types=[
            pltpu.VMEM((BLOCKS, 128), _i32),
            pltpu.VMEM((BLOCKS, 128), _i32),
            pltpu.VMEM((128, 16), _f32),
            pltpu.VMEM((128, 16), _f32),
            pltpu.VMEM((128, 16), _f32),
            pltpu.VMEM((128, 32), _f32),
            pltpu.VMEM_SHARED((NPAD, 32), _f32),
            pltpu.SemaphoreType.DMA,
            pltpu.SemaphoreType.DMA,
            pltpu.SemaphoreType.DMA,
        ],
    )(src2d, dst2d, h2, a2)

    out = pl.pallas_call(
        _tc_e,
        grid=(grid,),
        in_specs=[
            pl.BlockSpec((2, BLK, 32), lambda i: (0, i, 0)),
            pl.BlockSpec((1, NCLS), lambda i: (0, 0)),
        ],
        out_specs=pl.BlockSpec((BLK, NCLS), lambda i: (i, 0)),
        out_shape=jax.ShapeDtypeStruct((NPAD, NCLS), _f32),
    )(acc2, b2.reshape(1, NCLS).astype(_f32))

    return out[:N]
